# Initial kernel scaffold; baseline (speedup 1.0000x reference)
import jax
import jax.numpy as jnp
from jax.experimental import pallas as pl


def kernel(ids, table):
    raise NotImplementedError("write your pallas kernel here")


# speedup vs baseline: 3.1783x; 3.1783x over previous
import functools

import jax
import jax.numpy as jnp
from jax import lax
from jax.experimental import pallas as pl
from jax.experimental.pallas import tpu as pltpu
from jax.experimental.pallas import tpu_sc as plsc

NC = 2
NS = 16
NW = NC * NS

G = 128


@functools.partial(jax.jit, static_argnums=())
def kernel(ids, table):
    S, T = ids.shape
    V, D = table.shape
    B = S * T
    assert B % (NW * G) == 0
    b_per_w = B // NW
    n_g = b_per_w // G

    mesh = plsc.VectorSubcoreMesh(
        core_axis_name="c", subcore_axis_name="s",
        num_cores=NC, num_subcores=NS,
    )

    @functools.partial(
        pl.kernel,
        out_type=jax.ShapeDtypeStruct((B, D), jnp.float32),
        mesh=mesh,
        compiler_params=pltpu.CompilerParams(use_tc_tiling_on_sc=False),
        scratch_types=[
            pltpu.VMEM((G,), jnp.int32),
            pltpu.VMEM((G, D), jnp.float32),
            pltpu.SemaphoreType.DMA,
        ],
    )
    def gather_kernel(ids_hbm, table_hbm, out_hbm, idx_v, rows_v, sem):
        wid = lax.axis_index("s") * NC + lax.axis_index("c")
        base = wid * b_per_w

        def body(g, carry):
            off = base + g * G
            pltpu.sync_copy(ids_hbm.at[pl.ds(off, G)], idx_v)
            pltpu.async_copy(table_hbm.at[idx_v], rows_v, sem).wait()
            pltpu.sync_copy(rows_v, out_hbm.at[pl.ds(off, G)])
            return carry

        lax.fori_loop(0, n_g, body, 0)

    out = gather_kernel(ids.reshape(B), table)
    return out.reshape(S, T, D)


# speedup vs baseline: 4.2382x; 1.3334x over previous
import functools

import jax
import jax.numpy as jnp
from jax import lax
from jax.experimental import pallas as pl
from jax.experimental.pallas import tpu as pltpu
from jax.experimental.pallas import tpu_sc as plsc

NC = 2
NS = 16
NW = NC * NS

G = 128
K = 4


@functools.partial(jax.jit, static_argnums=())
def kernel(ids, table):
    S, T = ids.shape
    V, D = table.shape
    B = S * T
    assert B % (NW * G * K * 2) == 0
    b_per_w = B // NW
    n_g = b_per_w // G
    n_super = n_g // K
    n_pair = n_super // 2

    mesh = plsc.VectorSubcoreMesh(
        core_axis_name="c", subcore_axis_name="s",
        num_cores=NC, num_subcores=NS,
    )

    @functools.partial(
        pl.kernel,
        out_type=jax.ShapeDtypeStruct((B, D), jnp.float32),
        mesh=mesh,
        compiler_params=pltpu.CompilerParams(use_tc_tiling_on_sc=False),
        scratch_types=[
            pltpu.VMEM((n_g, G), jnp.int32),
            pltpu.VMEM((K * G, D), jnp.float32),
            pltpu.VMEM((K * G, D), jnp.float32),
            pltpu.SemaphoreType.DMA,
            pltpu.SemaphoreType.DMA,
            pltpu.SemaphoreType.DMA,
            pltpu.SemaphoreType.DMA,
        ],
    )
    def gather_kernel(ids_hbm, table_hbm, out_hbm,
                      idx_v, rows_a, rows_b, sga, sgb, swa, swb):
        wid = lax.axis_index("s") * NC + lax.axis_index("c")
        base = wid * b_per_w

        pltpu.sync_copy(ids_hbm.at[wid], idx_v)

        def superchunk(t, s, rows, sg, sw):
            @pl.when(t > 0)
            def _():
                pltpu.make_async_copy(
                    rows, out_hbm.at[pl.ds(base, K * G)], sw).wait()
            g0 = s * K
            waits = []
            for b in range(K):
                waits.append(pltpu.async_copy(
                    table_hbm.at[idx_v.at[g0 + b]],
                    rows.at[pl.ds(b * G, G)], sg))
            for w in waits:
                w.wait()
            pltpu.async_copy(
                rows, out_hbm.at[pl.ds(base + g0 * G, K * G)], sw)

        def body(t, carry):
            superchunk(t, 2 * t, rows_a, sga, swa)
            superchunk(t, 2 * t + 1, rows_b, sgb, swb)
            return carry

        lax.fori_loop(0, n_pair, body, 0)

        pltpu.make_async_copy(rows_a, out_hbm.at[pl.ds(base, K * G)], swa).wait()
        pltpu.make_async_copy(rows_b, out_hbm.at[pl.ds(base, K * G)], swb).wait()

    out = gather_kernel(ids.reshape(NW, n_g, G), table)
    return out.reshape(S, T, D)
